# baseline (speedup 1.0000x reference)
import jax
import jax.numpy as jnp
from jax import lax
from jax.experimental import pallas as pl
from jax.experimental.pallas import tpu as pltpu
from jax.experimental.pallas import tpu_sc as plsc

VOCAB = 1000000
N_EMBD = 64
BLOCK = 256
B = 1024
T = 256

NC = 2
NS = 16
NW = NC * NS

ROWS = B * T
R_PER_W = ROWS // NW
CHUNK = 128
NCHUNK = R_PER_W // CHUNK
NSLOT = 4
NOBUF = 2
OROWS = CHUNK // 2


def _emb_body(idx_hbm, tok_hbm, pos_hbm, out_hbm, idx_v, idx2_v, pos_v,
              rows_v, out_st, *sems):
    g_sem = sems[:NSLOT]
    o_sem = sems[NSLOT:]
    wid = lax.axis_index("s") * NC + lax.axis_index("c")
    obase = wid * (R_PER_W // 2)

    pltpu.sync_copy(idx_hbm.at[wid], idx_v)
    pltpu.sync_copy(pos_hbm, pos_v)

    def shift_body(i, _):
        j = i // (CHUNK // 16)
        k = (i % (CHUNK // 16)) * 16
        idx2_v[j, pl.ds(k, 16)] = jax.lax.shift_right_logical(
            idx_v[j, pl.ds(k, 16)], 1
        )
        return 0

    lax.fori_loop(0, NCHUNK * (CHUNK // 16), shift_body, 0, unroll=4)

    def start_gather(c, s):
        pltpu.async_copy(tok_hbm.at[idx2_v.at[c]], rows_v.at[s], g_sem[s])

    def wait_gather(s):
        pltpu.make_async_copy(
            tok_hbm.at[idx2_v.at[0]], rows_v.at[s], g_sem[s]
        ).wait()

    def start_out(c, b):
        pltpu.async_copy(
            out_st.at[b], out_hbm.at[pl.ds(obase + c * OROWS, OROWS)], o_sem[b]
        )

    def wait_out(b):
        pltpu.make_async_copy(
            out_st.at[b], out_hbm.at[pl.ds(obase, OROWS)], o_sem[b]
        ).wait()

    def extract(c, s, b):
        t0 = b * CHUNK

        def grp_body(k, _):
            v16 = idx_v[c, pl.ds(k * 16, 16)]
            for l in range(16):
                off = (v16[l] & 1) << 6
                r = k * 16 + l
                orow = k * 8 + l // 2
                pr = t0 // 2 + k * 8 + l // 2
                col = (l % 2) << 6
                for q in range(N_EMBD // 16):
                    out_st[b, orow, pl.ds(col + q * 16, 16)] = (
                        rows_v[s, r, pl.ds(off + q * 16, 16)]
                        + pos_v[pr, pl.ds(col + q * 16, 16)]
                    )
            return 0

        lax.fori_loop(0, CHUNK // 16, grp_body, 0)

    for s in range(NSLOT):
        start_gather(s, s)

    for cc in range(NSLOT):
        wait_gather(cc)
        if cc >= NOBUF:
            wait_out(cc % NOBUF)
        extract(cc, cc, cc % NOBUF)
        start_out(cc, cc % NOBUF)
        start_gather(cc + NSLOT, cc)

    def group(jo, _):
        for cc in range(NSLOT):
            c = jo * NSLOT + cc
            wait_gather(cc)
            wait_out(cc % NOBUF)
            extract(c, cc, cc % NOBUF)
            start_out(c, cc % NOBUF)
            start_gather(c + NSLOT, cc)
        return 0

    lax.fori_loop(1, NCHUNK // NSLOT - 1, group, 0)

    for cc in range(NSLOT):
        c = NCHUNK - NSLOT + cc
        wait_gather(cc)
        wait_out(cc % NOBUF)
        extract(c, cc, cc % NOBUF)
        start_out(c, cc % NOBUF)

    for b in range(NOBUF):
        wait_out(b)


@jax.jit
def _emb_call(idx, tok2, pos2):
    mesh = plsc.VectorSubcoreMesh(
        core_axis_name="c", subcore_axis_name="s", num_cores=NC, num_subcores=NS
    )
    return pl.kernel(
        _emb_body,
        out_type=jax.ShapeDtypeStruct((ROWS // 2, 2 * N_EMBD), jnp.float32),
        mesh=mesh,
        scratch_types=[
            pltpu.VMEM((NCHUNK, CHUNK), jnp.int32),
            pltpu.VMEM((NCHUNK, CHUNK), jnp.int32),
            pltpu.VMEM((BLOCK // 2, 2 * N_EMBD), jnp.float32),
            pltpu.VMEM((NSLOT, CHUNK, 2 * N_EMBD), jnp.float32),
            pltpu.VMEM((NOBUF, OROWS, 2 * N_EMBD), jnp.float32),
        ]
        + [pltpu.SemaphoreType.DMA] * (NSLOT + NOBUF),
        compiler_params=pltpu.CompilerParams(use_tc_tiling_on_sc=True),
    )(idx, tok2, pos2)


def kernel(x, tok_table, pos_table):
    idx = x.astype(jnp.int32).reshape(NW, NCHUNK, CHUNK)
    tok2 = tok_table.reshape(VOCAB // 2, 2 * N_EMBD)
    pos2 = pos_table.reshape(BLOCK // 2, 2 * N_EMBD)
    out = _emb_call(idx, tok2, pos2)
    return out.reshape(B, T, N_EMBD)


# speedup vs baseline: 1.0875x; 1.0875x over previous
import jax
import jax.numpy as jnp
from jax import lax
from jax.experimental import pallas as pl
from jax.experimental.pallas import tpu as pltpu
from jax.experimental.pallas import tpu_sc as plsc

VOCAB = 1000000
N_EMBD = 64
BLOCK = 256
B = 1024
T = 256

NC = 2
NS = 16
NW = NC * NS

ROWS = B * T
R_PER_W = ROWS // NW
SEQ_PER_W = R_PER_W // T
CHUNK = 128
NCHUNK = R_PER_W // CHUNK
NSLOT = 4
NOBUF = 2


def _emb_body(idx_hbm, tok_hbm, pos_hbm, out_hbm, idx_v, pos_v, rows_v,
              out_st, *sems):
    g_sem = sems[:NSLOT]
    o_sem = sems[NSLOT:]
    wid = lax.axis_index("s") * NC + lax.axis_index("c")
    sbase = wid * SEQ_PER_W

    pltpu.sync_copy(idx_hbm.at[wid], idx_v)
    pltpu.sync_copy(pos_hbm, pos_v)

    def start_gather(c, s):
        pltpu.async_copy(tok_hbm.at[idx_v.at[c]], rows_v.at[s], g_sem[s])

    def wait_gather(s):
        pltpu.make_async_copy(
            tok_hbm.at[idx_v.at[0]], rows_v.at[s], g_sem[s]
        ).wait()

    def start_out(c, b):
        pltpu.async_copy(
            out_st.at[b],
            out_hbm.at[sbase + c // 2, pl.ds(b * CHUNK, CHUNK)],
            o_sem[b],
        )

    def wait_out(b):
        pltpu.make_async_copy(
            out_st.at[b], out_hbm.at[sbase, pl.ds(b * CHUNK, CHUNK)], o_sem[b]
        ).wait()

    def add_pos(s, b):
        def row_body(r, _):
            for q in range(N_EMBD // 16):
                sl = pl.ds(q * 16, 16)
                out_st[b, r, sl] = rows_v[s, r, sl] + pos_v[b * CHUNK + r, sl]
            return 0

        lax.fori_loop(0, CHUNK, row_body, 0, unroll=4)

    for s in range(NSLOT):
        start_gather(s, s)

    for cc in range(NSLOT):
        wait_gather(cc)
        if cc >= NOBUF:
            wait_out(cc % NOBUF)
        add_pos(cc, cc % NOBUF)
        start_out(cc, cc % NOBUF)
        start_gather(cc + NSLOT, cc)

    def group(jo, _):
        for cc in range(NSLOT):
            c = jo * NSLOT + cc
            wait_gather(cc)
            wait_out(cc % NOBUF)
            add_pos(cc, cc % NOBUF)
            start_out(c, cc % NOBUF)
            start_gather(c + NSLOT, cc)
        return 0

    lax.fori_loop(1, NCHUNK // NSLOT - 1, group, 0)

    for cc in range(NSLOT):
        c = NCHUNK - NSLOT + cc
        wait_gather(cc)
        wait_out(cc % NOBUF)
        add_pos(cc, cc % NOBUF)
        start_out(c, cc % NOBUF)

    for b in range(NOBUF):
        wait_out(b)


@jax.jit
def _emb_call(idx, tok_table, pos_table):
    mesh = plsc.VectorSubcoreMesh(
        core_axis_name="c", subcore_axis_name="s", num_cores=NC, num_subcores=NS
    )
    return pl.kernel(
        _emb_body,
        out_type=jax.ShapeDtypeStruct((B, T, N_EMBD), jnp.float32),
        mesh=mesh,
        scratch_types=[
            pltpu.VMEM((NCHUNK, CHUNK), jnp.int32),
            pltpu.VMEM((BLOCK, N_EMBD), jnp.float32),
            pltpu.VMEM((NSLOT, CHUNK, N_EMBD), jnp.float32),
            pltpu.VMEM((NOBUF, CHUNK, N_EMBD), jnp.float32),
        ]
        + [pltpu.SemaphoreType.DMA] * (NSLOT + NOBUF),
        compiler_params=pltpu.CompilerParams(use_tc_tiling_on_sc=False),
    )(idx, tok_table, pos_table)


def kernel(x, tok_table, pos_table):
    idx = x.astype(jnp.int32).reshape(NW, NCHUNK, CHUNK)
    return _emb_call(idx, tok_table, pos_table)


# speedup vs baseline: 1.1598x; 1.0665x over previous
import jax
import jax.numpy as jnp
from jax import lax
from jax.experimental import pallas as pl
from jax.experimental.pallas import tpu as pltpu
from jax.experimental.pallas import tpu_sc as plsc

VOCAB = 1000000
N_EMBD = 64
BLOCK = 256
B = 1024
T = 256

NC = 2
NS = 16
NW = NC * NS

ROWS = B * T
R_PER_W = ROWS // NW
SEQ_PER_W = R_PER_W // T
CHUNK = 128
NCHUNK = R_PER_W // CHUNK
NSLOT = 8
LOOK = 4


def _emb_body(idx_hbm, tok_hbm, pos_hbm, out_hbm, idx_v, pos_v, st, *sems):
    g_sem = sems[:NSLOT]
    o_sem = sems[NSLOT:]
    wid = lax.axis_index("s") * NC + lax.axis_index("c")
    sbase = wid * SEQ_PER_W

    pltpu.sync_copy(idx_hbm.at[wid], idx_v)
    pltpu.sync_copy(pos_hbm, pos_v)

    def refill(s):
        t0 = (s % 2) * CHUNK

        def row_body(r, _):
            for q in range(N_EMBD // 16):
                sl = pl.ds(q * 16, 16)
                st[s, r, sl] = pos_v[t0 + r, sl]
            return 0

        lax.fori_loop(0, CHUNK, row_body, 0, unroll=4)

    def start_gadd(c, s):
        pltpu.async_copy(tok_hbm.at[idx_v.at[c]], st.at[s], g_sem[s], add=True)

    def wait_gather(s):
        pltpu.make_async_copy(
            tok_hbm.at[idx_v.at[0]], st.at[s], g_sem[s]
        ).wait()

    def start_out(c, s):
        pltpu.async_copy(
            st.at[s],
            out_hbm.at[sbase + c // 2, pl.ds((s % 2) * CHUNK, CHUNK)],
            o_sem[s],
        )

    def wait_out(s):
        pltpu.make_async_copy(
            st.at[s], out_hbm.at[sbase, pl.ds(0, CHUNK)], o_sem[s]
        ).wait()

    for cc in range(LOOK):
        refill(cc)
        start_gadd(cc, cc)

    for cc in range(NSLOT):
        wait_gather(cc)
        start_out(cc, cc)
        p = cc + LOOK
        if p < NSLOT:
            refill(p)
            start_gadd(p, p)
        else:
            sp = p - NSLOT
            wait_out(sp)
            refill(sp)
            start_gadd(p, sp)

    def group(jo, _):
        for cc in range(NSLOT):
            c = jo * NSLOT + cc
            wait_gather(cc)
            start_out(c, cc)
            sp = (cc + LOOK) % NSLOT
            wait_out(sp)
            refill(sp)
            start_gadd(c + LOOK, sp)
        return 0

    lax.fori_loop(1, NCHUNK // NSLOT - 1, group, 0)

    for cc in range(NSLOT):
        c = NCHUNK - NSLOT + cc
        wait_gather(cc)
        start_out(c, cc)
        if cc < LOOK:
            sp = (cc + LOOK) % NSLOT
            wait_out(sp)
            refill(sp)
            start_gadd(c + LOOK, sp)

    for s in range(NSLOT):
        wait_out(s)


@jax.jit
def _emb_call(idx, tok_table, pos_table):
    mesh = plsc.VectorSubcoreMesh(
        core_axis_name="c", subcore_axis_name="s", num_cores=NC, num_subcores=NS
    )
    return pl.kernel(
        _emb_body,
        out_type=jax.ShapeDtypeStruct((B, T, N_EMBD), jnp.float32),
        mesh=mesh,
        scratch_types=[
            pltpu.VMEM((NCHUNK, CHUNK), jnp.int32),
            pltpu.VMEM((BLOCK, N_EMBD), jnp.float32),
            pltpu.VMEM((NSLOT, CHUNK, N_EMBD), jnp.float32),
        ]
        + [pltpu.SemaphoreType.DMA] * (2 * NSLOT),
        compiler_params=pltpu.CompilerParams(use_tc_tiling_on_sc=False),
    )(idx, tok_table, pos_table)


def kernel(x, tok_table, pos_table):
    idx = x.astype(jnp.int32).reshape(NW, NCHUNK, CHUNK)
    return _emb_call(idx, tok_table, pos_table)
